# baseline (speedup 1.0000x reference)
import functools

import jax
import jax.numpy as jnp
from jax import lax
from jax.experimental import pallas as pl
from jax.experimental.pallas import tpu as pltpu
from jax.experimental.pallas import tpu_sc as plsc

NUM_CORES = 2
NUM_SUBCORES = 16
NUM_WORKERS = NUM_CORES * NUM_SUBCORES
BLK = 128
NBUF = 4


@functools.partial(jax.jit, static_argnames=("total", "dim", "seq", "bpl", "nblk"))
def _gather_sc(idx4, table, *, total, dim, seq, bpl, nblk):
    mesh = plsc.VectorSubcoreMesh(
        core_axis_name="c", subcore_axis_name="s",
        num_cores=NUM_CORES, num_subcores=NUM_SUBCORES)

    @functools.partial(
        pl.kernel,
        out_type=jax.ShapeDtypeStruct((total + BLK, dim), table.dtype),
        mesh=mesh,
        compiler_params=pltpu.CompilerParams(use_tc_tiling_on_sc=False),
        scratch_types=[
            pltpu.VMEM((nblk, BLK), jnp.int32),
            pltpu.VMEM((NBUF, BLK, dim), table.dtype),
        ] + [pltpu.SemaphoreType.DMA] * (2 * NBUF),
    )
    def body(idx_hbm, table_hbm, out_hbm, idx_v, rows_v, *sems):
        gsems = sems[:NBUF]
        wsems = sems[NBUF:]
        wid = lax.axis_index("s") * NUM_CORES + lax.axis_index("c")
        pltpu.sync_copy(idx_hbm.at[wid], idx_v)

        def gather(j, bb):
            return pltpu.make_async_copy(
                table_hbm.at[idx_v.at[j]], rows_v.at[bb], gsems[bb])

        def wb(j, bb):
            p = wid * nblk + j
            tr = p // (bpl * 8)
            rm = p % (bpl * 8)
            tc = rm // 8
            s = rm % 8
            l = tr * 8 + s
            base = jnp.where(l < seq, (l * bpl + tc) * BLK, total)
            return pltpu.make_async_copy(
                rows_v.at[bb], out_hbm.at[pl.ds(base, BLK)], wsems[bb])

        gather(0, 0).start()
        gather(1, 1).start()

        def step(go, carry):
            for bb in range(NBUF):
                j = go * NBUF + bb
                gather(j, bb).wait()
                wb(j, bb).start()

                @pl.when(j + 2 < nblk)
                def _():
                    bn = (bb + 2) % NBUF

                    @pl.when(j - 2 >= 0)
                    def _():
                        wb(j - 2, bn).wait()

                    gather(j + 2, bn).start()
            return carry

        lax.fori_loop(0, nblk // NBUF, step, 0)
        for bb in range(NBUF):
            wb(nblk - NBUF + bb, bb).wait()

    return body(idx4, table)


def kernel(indices, table):
    bsz, seq = indices.shape
    dim = table.shape[1]
    total = bsz * seq
    seq_p = ((seq + 7) // 8) * 8
    ntr = seq_p // 8
    bpl = bsz // BLK
    assert bsz % BLK == 0
    nblocks = ntr * bpl * 8
    assert nblocks % (NUM_WORKERS * NBUF) == 0
    nblk = nblocks // NUM_WORKERS
    idx_p = jnp.pad(indices.astype(jnp.int32), ((0, 0), (0, seq_p - seq)))
    idx4 = (idx_p.T.reshape(ntr, 8, bpl, BLK)
            .transpose(0, 2, 1, 3)
            .reshape(NUM_WORKERS, nblk, BLK))
    out = _gather_sc(idx4, table, total=total, dim=dim, seq=seq,
                     bpl=bpl, nblk=nblk)
    return out[:total].reshape(seq, bsz, dim).transpose(1, 0, 2)
